# baseline (speedup 1.0000x reference)
import functools

import jax
import jax.numpy as jnp
from jax import lax
from jax.experimental import pallas as pl
from jax.experimental.pallas import tpu as pltpu
from jax.experimental.pallas import tpu_sc as plsc

CODEBOOK_SIZE = 8192
CODEBOOK_DIM = 256
N_TOKENS = 262144

NUM_CORES = 2
NUM_SUBCORES = 16
NUM_WORKERS = NUM_CORES * NUM_SUBCORES
B_PER_W = N_TOKENS // NUM_WORKERS
CHUNK = 64
NCHUNK = B_PER_W // CHUNK
NBUF = 3

_MESH = plsc.VectorSubcoreMesh(core_axis_name="c", subcore_axis_name="s")


@functools.partial(
    pl.kernel,
    mesh=_MESH,
    out_type=jax.ShapeDtypeStruct((N_TOKENS, CODEBOOK_DIM), jnp.float32),
    scratch_types=[
        pltpu.VMEM((NCHUNK, CHUNK), jnp.int32),
        pltpu.VMEM((NBUF, CHUNK, CODEBOOK_DIM), jnp.float32),
        pltpu.VMEM_SHARED((NUM_SUBCORES, NBUF, CHUNK, CODEBOOK_DIM),
                          jnp.float32),
        pltpu.SemaphoreType.DMA,
        pltpu.SemaphoreType.DMA,
        pltpu.SemaphoreType.DMA,
        pltpu.SemaphoreType.DMA,
        pltpu.SemaphoreType.DMA,
        pltpu.SemaphoreType.DMA,
        pltpu.SemaphoreType.DMA,
        pltpu.SemaphoreType.DMA,
        pltpu.SemaphoreType.DMA,
    ],
)
def _codebook_gather(weight_hbm, idx_hbm, out_hbm, idx_v, rows_v, spm,
                     gsem0, gsem1, gsem2, xsem0, xsem1, xsem2,
                     dsem0, dsem1, dsem2):
    s = lax.axis_index("s")
    wid = s * NUM_CORES + lax.axis_index("c")
    base = wid * B_PER_W
    gsems = [gsem0, gsem1, gsem2]
    xsems = [xsem0, xsem1, xsem2]
    dsems = [dsem0, dsem1, dsem2]

    pltpu.sync_copy(idx_hbm.at[wid], idx_v)

    def start_gather(g, b):
        pltpu.make_async_copy(
            weight_hbm.at[idx_v.at[g]], rows_v.at[b], gsems[b]).start()

    def wait_gather(b):
        pltpu.make_async_copy(
            weight_hbm.at[idx_v.at[0]], rows_v.at[b], gsems[b]).wait()

    def start_xcopy(b):
        pltpu.make_async_copy(rows_v.at[b], spm.at[s, b], xsems[b]).start()

    def wait_xcopy(b):
        pltpu.make_async_copy(rows_v.at[b], spm.at[s, b], xsems[b]).wait()

    def start_drain(g, b):
        pltpu.make_async_copy(
            spm.at[s, b], out_hbm.at[pl.ds(base + g * CHUNK, CHUNK)],
            dsems[b]).start()

    def wait_drain(b):
        pltpu.make_async_copy(
            spm.at[s, b], out_hbm.at[pl.ds(base, CHUNK)], dsems[b]).wait()

    start_gather(0, 0)
    start_gather(1, 1)
    wait_gather(0)
    start_xcopy(0)
    start_gather(2, 2)
    wait_gather(1)
    start_xcopy(1)
    wait_xcopy(0)
    start_gather(3, 0)
    start_drain(0, 0)

    def steady(i, carry):
        for j in range(NBUF):
            g = 2 + NBUF * i + j
            b = (2 + j) % NBUF
            pm1 = (1 + j) % NBUF
            pm2 = j % NBUF
            wait_gather(b)
            start_xcopy(b)
            wait_xcopy(pm1)
            start_gather(g + 2, pm1)
            start_drain(g - 1, pm1)
            wait_drain(pm2)
        return carry

    lax.fori_loop(0, (NCHUNK - 5) // NBUF, steady, 0)

    g = NCHUNK - 3
    wait_gather(2)
    start_xcopy(2)
    wait_xcopy(1)
    start_gather(g + 2, 1)
    start_drain(g - 1, 1)
    wait_drain(0)
    g = NCHUNK - 2
    wait_gather(0)
    start_xcopy(0)
    wait_xcopy(2)
    start_drain(g - 1, 2)
    wait_drain(1)
    g = NCHUNK - 1
    wait_gather(1)
    start_xcopy(1)
    wait_xcopy(0)
    start_drain(g - 1, 0)
    wait_drain(2)
    wait_xcopy(1)
    start_drain(NCHUNK - 1, 1)
    wait_drain(0)
    wait_drain(1)


def kernel(embed_id, weight):
    idx = embed_id.astype(jnp.int32).reshape(NUM_WORKERS, NCHUNK, CHUNK)
    return _codebook_gather(weight, idx)
